# Initial kernel scaffold; baseline (speedup 1.0000x reference)
import jax
import jax.numpy as jnp
from jax.experimental import pallas as pl


def kernel(updates, mask):
    raise NotImplementedError("write your pallas kernel here")


# speedup vs baseline: 18.4697x; 18.4697x over previous
import functools

import jax
import jax.numpy as jnp
from jax import lax
from jax.experimental import pallas as pl
from jax.experimental.pallas import tpu as pltpu
from jax.experimental.pallas import tpu_sc as plsc

_B, _H, _W, _C = 8, 112, 112, 192
_HW = _H * _W
_P = _HW * 4
_NPAIR = _B * _C
_NW = 32
_PPW = _NPAIR // _NW
_L = 16


def _sc_unpool_scatter(u_t, m_t):
    mesh = plsc.VectorSubcoreMesh(core_axis_name="c", subcore_axis_name="s")

    @functools.partial(
        pl.kernel,
        out_type=jax.ShapeDtypeStruct((_NPAIR, _P), jnp.float32),
        mesh=mesh,
        compiler_params=pltpu.CompilerParams(needs_layout_passes=False),
        scratch_types=[
            pltpu.VMEM((_HW,), jnp.float32),
            pltpu.VMEM((_HW,), jnp.int32),
            pltpu.VMEM((_P,), jnp.float32),
        ],
    )
    def k(u_hbm, m_hbm, out_hbm, vals, msk, acc):
        wid = lax.axis_index("s") * 2 + lax.axis_index("c")

        def pair_body(j, carry):
            pr = wid * _PPW + j
            pltpu.sync_copy(u_hbm.at[pr], vals)
            pltpu.sync_copy(m_hbm.at[pr], msk)

            def zbody(i, c):
                acc[pl.ds(i * _L, _L)] = jnp.zeros((_L,), jnp.float32)
                return c

            lax.fori_loop(0, _P // _L, zbody, 0)

            def sbody(i, c):
                m = msk[pl.ds(i * _L, _L)]
                v = vals[pl.ds(i * _L, _L)]
                p = lax.div(m, jnp.full((_L,), _C, jnp.int32))
                plsc.addupdate_scatter(acc, [p], v)
                return c

            lax.fori_loop(0, _HW // _L, sbody, 0)

            pltpu.sync_copy(acc, out_hbm.at[pr])
            return carry

        lax.fori_loop(0, _PPW, pair_body, 0)

    return k(u_t, m_t)


def kernel(updates, mask):
    u_t = jnp.transpose(updates, (0, 3, 1, 2)).reshape(_NPAIR, _HW)
    m_t = jnp.transpose(mask, (0, 3, 1, 2)).reshape(_NPAIR, _HW)
    out_t = _sc_unpool_scatter(u_t, m_t)
    return out_t.reshape(_B, _C, _H * 2, _W * 2).transpose(0, 2, 3, 1)


# speedup vs baseline: 23.6694x; 1.2815x over previous
import functools

import jax
import jax.numpy as jnp
from jax import lax
from jax.experimental import pallas as pl
from jax.experimental.pallas import tpu as pltpu
from jax.experimental.pallas import tpu_sc as plsc

_B, _H, _W, _C = 8, 112, 112, 192
_HW = _H * _W
_P = _HW * 4
_NPAIR = _B * _C
_NW = 32
_PPW = _NPAIR // _NW
_L = 16
_ZU = 16
_SU = 8


def _sc_unpool_scatter(u_t, m_t):
    mesh = plsc.VectorSubcoreMesh(core_axis_name="c", subcore_axis_name="s")

    @functools.partial(
        pl.kernel,
        out_type=jax.ShapeDtypeStruct((_NPAIR, _P), jnp.float32),
        mesh=mesh,
        compiler_params=pltpu.CompilerParams(needs_layout_passes=False),
        scratch_types=[
            pltpu.VMEM((_HW,), jnp.float32),
            pltpu.VMEM((_HW,), jnp.float32),
            pltpu.VMEM((_HW,), jnp.int32),
            pltpu.VMEM((_HW,), jnp.int32),
            pltpu.VMEM((_P,), jnp.float32),
            pltpu.SemaphoreType.DMA,
            pltpu.SemaphoreType.DMA,
        ],
    )
    def k(u_hbm, m_hbm, out_hbm, vals_a, vals_b, msk_a, msk_b, acc,
          sem_a, sem_b):
        wid = lax.axis_index("s") * 2 + lax.axis_index("c")
        base_pr = wid * _PPW

        def fetch(pr, vals, msk, sem):
            prc = jnp.minimum(pr, _NPAIR - 1)
            cp_v = pltpu.make_async_copy(u_hbm.at[prc], vals, sem)
            cp_m = pltpu.make_async_copy(m_hbm.at[prc], msk, sem)
            cp_v.start()
            cp_m.start()
            return cp_v, cp_m

        def process(pr, vals, msk, sem, nvals, nmsk, nsem):
            fetch(pr + 1, nvals, nmsk, nsem)

            def zbody(i, c):
                b0 = i * (_ZU * _L)
                for u in range(_ZU):
                    acc[pl.ds(b0 + u * _L, _L)] = jnp.zeros((_L,),
                                                            jnp.float32)
                return c

            lax.fori_loop(0, _P // (_ZU * _L), zbody, 0, unroll=False)

            pltpu.make_async_copy(u_hbm.at[0], vals, sem).wait()
            pltpu.make_async_copy(m_hbm.at[0], msk, sem).wait()

            def sbody(i, c):
                b0 = i * (_SU * _L)
                for u in range(_SU):
                    off = b0 + u * _L
                    m = msk[pl.ds(off, _L)]
                    v = vals[pl.ds(off, _L)]
                    p = lax.div(m, jnp.full((_L,), _C, jnp.int32))
                    plsc.addupdate_scatter(acc, [p], v)
                return c

            lax.fori_loop(0, _HW // (_SU * _L), sbody, 0, unroll=False)

            pltpu.sync_copy(acc, out_hbm.at[pr])

        fetch(base_pr, vals_a, msk_a, sem_a)

        def pair_body(j, carry):
            pr = base_pr + 2 * j
            process(pr, vals_a, msk_a, sem_a, vals_b, msk_b, sem_b)
            process(pr + 1, vals_b, msk_b, sem_b, vals_a, msk_a, sem_a)
            return carry

        lax.fori_loop(0, _PPW // 2, pair_body, 0)

        pltpu.make_async_copy(u_hbm.at[0], vals_a, sem_a).wait()
        pltpu.make_async_copy(m_hbm.at[0], msk_a, sem_a).wait()

    return k(u_t, m_t)


def kernel(updates, mask):
    u_t = jnp.transpose(updates, (0, 3, 1, 2)).reshape(_NPAIR, _HW)
    m_t = jnp.transpose(mask, (0, 3, 1, 2)).reshape(_NPAIR, _HW)
    out_t = _sc_unpool_scatter(u_t, m_t)
    return out_t.reshape(_B, _C, _H * 2, _W * 2).transpose(0, 2, 3, 1)


# speedup vs baseline: 55.8706x; 2.3605x over previous
import functools

import jax
import jax.numpy as jnp
from jax import lax
from jax.experimental import pallas as pl
from jax.experimental.pallas import tpu as pltpu
from jax.experimental.pallas import tpu_sc as plsc

_B, _H, _W, _C = 8, 112, 112, 192
_HW = _H * _W
_P = _HW * 4
_NPAIR = _B * _C
_NW = 32
_PPW = _NPAIR // _NW
_L = 16
_ZU = 16
_SU = 8


def _sc_unpool_scatter(u_t, m_t):
    mesh = plsc.VectorSubcoreMesh(core_axis_name="c", subcore_axis_name="s")

    @functools.partial(
        pl.kernel,
        out_type=jax.ShapeDtypeStruct((_NPAIR, _P), jnp.float32),
        mesh=mesh,
        compiler_params=pltpu.CompilerParams(needs_layout_passes=False),
        scratch_types=[
            pltpu.VMEM((_HW,), jnp.float32),
            pltpu.VMEM((_HW,), jnp.float32),
            pltpu.VMEM((_HW,), jnp.int32),
            pltpu.VMEM((_HW,), jnp.int32),
            pltpu.VMEM((_P,), jnp.float32),
            pltpu.SemaphoreType.DMA,
            pltpu.SemaphoreType.DMA,
        ],
    )
    def k(u_hbm, m_hbm, out_hbm, vals_a, vals_b, msk_a, msk_b, acc,
          sem_a, sem_b):
        wid = lax.axis_index("s") * 2 + lax.axis_index("c")
        base_pr = wid * _PPW

        def fetch(pr, vals, msk, sem):
            prc = jnp.minimum(pr, _NPAIR - 1)
            cp_v = pltpu.make_async_copy(u_hbm.at[prc], vals, sem)
            cp_m = pltpu.make_async_copy(m_hbm.at[prc], msk, sem)
            cp_v.start()
            cp_m.start()
            return cp_v, cp_m

        def process(pr, vals, msk, sem, nvals, nmsk, nsem):
            fetch(pr + 1, nvals, nmsk, nsem)

            @plsc.parallel_loop(0, _P // _L, unroll=_ZU)
            def zbody(i):
                acc[pl.ds(i * _L, _L)] = jnp.zeros((_L,), jnp.float32)

            pltpu.make_async_copy(u_hbm.at[0], vals, sem).wait()
            pltpu.make_async_copy(m_hbm.at[0], msk, sem).wait()

            @plsc.parallel_loop(0, _HW // _L, unroll=_SU)
            def sbody(i):
                off = i * _L
                m = msk[pl.ds(off, _L)]
                v = vals[pl.ds(off, _L)]
                t = lax.shift_right_logical(m, 6)
                p = (t.astype(jnp.float32) *
                     jnp.float32(1.0 / 3.0)).astype(jnp.int32)
                plsc.addupdate_scatter(acc, [p], v)

            pltpu.sync_copy(acc, out_hbm.at[pr])

        fetch(base_pr, vals_a, msk_a, sem_a)

        def pair_body(j, carry):
            pr = base_pr + 2 * j
            process(pr, vals_a, msk_a, sem_a, vals_b, msk_b, sem_b)
            process(pr + 1, vals_b, msk_b, sem_b, vals_a, msk_a, sem_a)
            return carry

        lax.fori_loop(0, _PPW // 2, pair_body, 0)

        pltpu.make_async_copy(u_hbm.at[0], vals_a, sem_a).wait()
        pltpu.make_async_copy(m_hbm.at[0], msk_a, sem_a).wait()

    return k(u_t, m_t)


def kernel(updates, mask):
    u_t = jnp.transpose(updates, (0, 3, 1, 2)).reshape(_NPAIR, _HW)
    m_t = jnp.transpose(mask, (0, 3, 1, 2)).reshape(_NPAIR, _HW)
    out_t = _sc_unpool_scatter(u_t, m_t)
    return out_t.reshape(_B, _C, _H * 2, _W * 2).transpose(0, 2, 3, 1)


# speedup vs baseline: 80.4004x; 1.4390x over previous
import functools

import jax
import jax.numpy as jnp
from jax import lax
from jax.experimental import pallas as pl
from jax.experimental.pallas import tpu as pltpu
from jax.experimental.pallas import tpu_sc as plsc

_B, _H, _W, _C = 8, 112, 112, 192
_HW = _H * _W
_HO, _WO = _H * 2, _W * 2
_P = _HO * _WO
_NPAIR = _B * _C
_NW = 32
_PPW = _NPAIR // _NW
_WPB = _NW // _B
_L = 16
_ZU = 2
_SU = 2


def _sc_unpool_scatter(u_t, m_t):
    mesh = plsc.VectorSubcoreMesh(core_axis_name="c", subcore_axis_name="s")

    @functools.partial(
        pl.kernel,
        out_type=jax.ShapeDtypeStruct((_B, _C, _HO, _WO), jnp.float32),
        mesh=mesh,
        compiler_params=pltpu.CompilerParams(needs_layout_passes=False),
        scratch_types=[
            pltpu.VMEM((_H, _W), jnp.float32),
            pltpu.VMEM((_H, _W), jnp.float32),
            pltpu.VMEM((_H, _W), jnp.int32),
            pltpu.VMEM((_H, _W), jnp.int32),
            pltpu.VMEM((_HO, _WO), jnp.float32),
            pltpu.SemaphoreType.DMA,
            pltpu.SemaphoreType.DMA,
        ],
    )
    def k(u_hbm, m_hbm, out_hbm, vals_a, vals_b, msk_a, msk_b, acc,
          sem_a, sem_b):
        wid = lax.axis_index("s") * 2 + lax.axis_index("c")
        b = wid // _WPB
        c0 = (wid % _WPB) * _PPW

        def fetch(j, vals, msk, sem):
            c = c0 + jnp.minimum(j, _PPW - 1)
            pltpu.make_async_copy(u_hbm.at[b, c], vals, sem).start()
            pltpu.make_async_copy(m_hbm.at[b, c], msk, sem).start()

        def process(j, vals, msk, sem, nvals, nmsk, nsem):
            fetch(j + 1, nvals, nmsk, nsem)

            @plsc.parallel_loop(0, _HO, unroll=_ZU)
            def zbody(r):
                for g in range(_WO // _L):
                    acc[r, pl.ds(g * _L, _L)] = jnp.zeros((_L,), jnp.float32)

            pltpu.make_async_copy(u_hbm.at[0, 0], vals, sem).wait()
            pltpu.make_async_copy(m_hbm.at[0, 0], msk, sem).wait()

            @plsc.parallel_loop(0, _H, unroll=_SU)
            def sbody(r):
                for g in range(_W // _L):
                    m = msk[r, pl.ds(g * _L, _L)]
                    v = vals[r, pl.ds(g * _L, _L)]
                    t = lax.shift_right_logical(m, 6)
                    p = (t.astype(jnp.float32) *
                         jnp.float32(1.0 / 3.0)).astype(jnp.int32)
                    u = lax.shift_right_logical(m, 11)
                    py = (u.astype(jnp.float32) *
                          jnp.float32(1.0 / 21.0)).astype(jnp.int32)
                    px = p - py * _WO
                    plsc.addupdate_scatter(acc, [py, px], v)

            pltpu.sync_copy(acc, out_hbm.at[b, c0 + j])

        fetch(0, vals_a, msk_a, sem_a)

        def pair_body(i, carry):
            j = 2 * i
            process(j, vals_a, msk_a, sem_a, vals_b, msk_b, sem_b)
            process(j + 1, vals_b, msk_b, sem_b, vals_a, msk_a, sem_a)
            return carry

        lax.fori_loop(0, _PPW // 2, pair_body, 0)

        pltpu.make_async_copy(u_hbm.at[0, 0], vals_a, sem_a).wait()
        pltpu.make_async_copy(m_hbm.at[0, 0], msk_a, sem_a).wait()

    return k(u_t, m_t)


def kernel(updates, mask):
    u_t = jnp.transpose(updates, (0, 3, 1, 2))
    m_t = jnp.transpose(mask, (0, 3, 1, 2))
    out_t = _sc_unpool_scatter(u_t, m_t)
    return out_t.transpose(0, 2, 3, 1)
